# baseline (speedup 1.0000x reference)
import functools

import jax
import jax.numpy as jnp
from jax import lax
from jax.experimental import pallas as pl
from jax.experimental.pallas import tpu as pltpu
from jax.experimental.pallas import tpu_sc as plsc

VOCAB = 1000000
DIM = 32
NUM_LABELS = 10
BATCH = 4096
SEQ = 200

NC = 2
NS = 16
L = 16
NW = NC * NS
BPW = BATCH // NW
FLAT = BPW * SEQ
NBUF = 8
C1, C2 = 128, SEQ - 128
NCHUNK = 2 * BPW
SENT = -1

_mesh = plsc.VectorSubcoreMesh(core_axis_name="c", subcore_axis_name="s")


@functools.partial(
    pl.kernel,
    mesh=_mesh,
    out_type=[
        jax.ShapeDtypeStruct((BATCH, DIM), jnp.float32),
        jax.ShapeDtypeStruct((BATCH, L), jnp.float32),
    ],
    compiler_params=pltpu.CompilerParams(use_tc_tiling_on_sc=False),
    scratch_types=[
        pltpu.VMEM((FLAT,), jnp.int32),
        pltpu.VMEM((FLAT + L,), jnp.int32),
        pltpu.VMEM((NBUF, C1, DIM), jnp.float32),
        pltpu.VMEM((BPW, DIM), jnp.float32),
        pltpu.VMEM((NUM_LABELS, DIM), jnp.float32),
        pltpu.VMEM((L,), jnp.float32),
        pltpu.VMEM((BPW, L), jnp.float32),
    ] + [pltpu.SemaphoreType.DMA] * NBUF,
)
def _sc_pool(ids_hbm, mask_hbm, table_hbm, wt_hbm, b_hbm,
             out_hbm, out2_hbm,
             idv, mkv, ring, pooled_v, wt_v, b_v, logits_v, *sems):
    wid = lax.axis_index("s") * NC + lax.axis_index("c")
    base = wid * FLAT

    pltpu.sync_copy(ids_hbm.at[pl.ds(base, FLAT)], idv)
    pltpu.sync_copy(mask_hbm.at[pl.ds(base, FLAT)], mkv.at[pl.ds(0, FLAT)])
    pltpu.sync_copy(wt_hbm, wt_v)
    pltpu.sync_copy(b_hbm, b_v)

    zi = jnp.full((L,), 0, jnp.int32)
    one_i = zi + 1
    mkv[pl.ds(FLAT, L)] = zi

    MU = 8

    def _prep(i, carry):
        for k in range(MU):
            sl = pl.ds((i * MU + k) * L, L)
            idv[sl] = (idv[sl] + one_i) * mkv[sl] - one_i
        return carry

    lax.fori_loop(0, FLAT // (L * MU), _prep, 0)

    iot = lax.iota(jnp.int32, L)
    thresh = jnp.full((L,), SEQ % L, jnp.int32)
    lane = jnp.where(iot < thresh, one_i, zi)
    one_f = jnp.full((L,), 1.0, jnp.float32)
    zero_f = jnp.zeros((L,), jnp.float32)

    def _zero_slot(j, n):
        def _zb(i, carry):
            s0 = i * 8
            for k in range(8):
                ring[j, s0 + k, pl.ds(0, L)] = zero_f
                ring[j, s0 + k, pl.ds(L, L)] = zero_f
            return carry

        lax.fori_loop(0, n // 8, _zb, 0)

    def _start_chunk(rb, parity, j):
        if parity == 0:
            idx = plsc.Indices(idv.at[pl.ds(rb * SEQ, C1)],
                               ignored_value=SENT)
            return pltpu.async_copy(table_hbm.at[idx], ring.at[j], sems[j])
        idx = plsc.Indices(idv.at[pl.ds(rb * SEQ + C1, C2)],
                           ignored_value=SENT)
        return pltpu.async_copy(table_hbm.at[idx],
                                ring.at[j, pl.ds(0, C2), :], sems[j])

    def _accum(j, n, a0, a1):
        def _body(i, carry):
            b0, b1, b2, b3 = carry
            s0 = i * 8
            for k in range(8):
                lo = ring[j, s0 + k, pl.ds(0, L)]
                hi = ring[j, s0 + k, pl.ds(L, L)]
                if k % 2 == 0:
                    b0 = b0 + lo
                    b1 = b1 + hi
                else:
                    b2 = b2 + lo
                    b3 = b3 + hi
            return (b0, b1, b2, b3)

        b0, b1, b2, b3 = lax.fori_loop(0, n // 8, _body,
                                       (zero_f, zero_f, zero_f, zero_f))
        return a0 + b0 + b2, a1 + b1 + b3

    w_lo = [wt_v[l, pl.ds(0, L)] for l in range(NUM_LABELS)]
    w_hi = [wt_v[l, pl.ds(L, L)] for l in range(NUM_LABELS)]
    eqs = [iot == jnp.full((L,), l, jnp.int32) for l in range(NUM_LABELS)]
    bvec = b_v[pl.ds(0, L)]
    perms = [iot ^ jnp.full((L,), sh, jnp.int32) for sh in (8, 4, 2, 1)]

    def _finalize(rb, a0, a1):
        off = rb * SEQ
        cvec = mkv[pl.ds(off + (SEQ // L) * L, L)] * lane
        for k in range(SEQ // L):
            cvec = cvec + mkv[pl.ds(off + k * L, L)]
        for perm in perms:
            cvec = cvec + cvec.at[perm].get(mode="promise_in_bounds")
        inv = one_f / jnp.maximum(cvec.astype(jnp.float32), one_f)
        p0 = a0 * inv
        p1 = a1 * inv
        pooled_v[rb, pl.ds(0, L)] = p0
        pooled_v[rb, pl.ds(L, L)] = p1
        lacc = bvec
        for l in range(NUM_LABELS):
            t = p0 * w_lo[l] + p1 * w_hi[l]
            for perm in perms:
                t = t + t.at[perm].get(mode="promise_in_bounds")
            lacc = jnp.where(eqs[l], t + bvec, lacc)
        logits_v[rb, pl.ds(0, L)] = lacc

    for j in range(NBUF):
        _zero_slot(j, C1 if j % 2 == 0 else C2)
    handles = [_start_chunk(j // 2, j % 2, j) for j in range(NBUF)]

    def _outer(g, carry):
        c0 = g * NBUF
        for j in range(0, NBUF, 2):
            rb = c0 // 2 + j // 2
            handles[j].wait()
            a0, a1 = _accum(j, C1, zero_f, zero_f)

            @pl.when(c0 + NBUF + j < NCHUNK)
            def _():
                _zero_slot(j, C1)
                _start_chunk((c0 + NBUF + j) // 2, 0, j)

            handles[j + 1].wait()
            a0, a1 = _accum(j + 1, C2, a0, a1)

            @pl.when(c0 + NBUF + j + 1 < NCHUNK)
            def _():
                _zero_slot(j + 1, C2)
                _start_chunk((c0 + NBUF + j + 1) // 2, 1, j + 1)

            _finalize(rb, a0, a1)
        return carry

    lax.fori_loop(0, NCHUNK // NBUF, _outer, 0)

    pltpu.sync_copy(pooled_v, out_hbm.at[pl.ds(wid * BPW, BPW), :])
    pltpu.sync_copy(logits_v, out2_hbm.at[pl.ds(wid * BPW, BPW), :])


def kernel(input_ids, attention_mask, table, W, b):
    ids = input_ids.reshape(-1).astype(jnp.int32)
    mask = attention_mask.reshape(-1).astype(jnp.int32)
    b16 = jnp.pad(b, (0, L - NUM_LABELS))
    pooled, logits16 = _sc_pool(ids, mask, table, W.T, b16)
    return (logits16[:, :NUM_LABELS], pooled)


# speedup vs baseline: 1.0040x; 1.0040x over previous
import functools

import jax
import jax.numpy as jnp
from jax import lax
from jax.experimental import pallas as pl
from jax.experimental.pallas import tpu as pltpu
from jax.experimental.pallas import tpu_sc as plsc

VOCAB = 1000000
DIM = 32
NUM_LABELS = 10
BATCH = 4096
SEQ = 200

NC = 2
NS = 16
L = 16
NW = NC * NS
BPW = BATCH // NW
FLAT = BPW * SEQ
NBUF = 8
C1, C2 = 128, SEQ - 128
NCHUNK = 2 * BPW
SENT = -1

_mesh = plsc.VectorSubcoreMesh(core_axis_name="c", subcore_axis_name="s")


@functools.partial(
    pl.kernel,
    mesh=_mesh,
    out_type=jax.ShapeDtypeStruct((BATCH, DIM), jnp.float32),
    compiler_params=pltpu.CompilerParams(use_tc_tiling_on_sc=False),
    scratch_types=[
        pltpu.VMEM((FLAT,), jnp.int32),
        pltpu.VMEM((FLAT + L,), jnp.int32),
        pltpu.VMEM((NBUF, C1, DIM), jnp.float32),
        pltpu.VMEM((BPW, DIM), jnp.float32),
    ] + [pltpu.SemaphoreType.DMA] * NBUF,
)
def _sc_pool(ids_hbm, mask_hbm, table_hbm, out_hbm,
             idv, mkv, ring, pooled_v, *sems):
    wid = lax.axis_index("s") * NC + lax.axis_index("c")
    base = wid * FLAT

    pltpu.sync_copy(ids_hbm.at[pl.ds(base, FLAT)], idv)
    pltpu.sync_copy(mask_hbm.at[pl.ds(base, FLAT)], mkv.at[pl.ds(0, FLAT)])

    zi = jnp.full((L,), 0, jnp.int32)
    one_i = zi + 1
    mkv[pl.ds(FLAT, L)] = zi

    MU = 8

    def _prep(i, carry):
        for k in range(MU):
            sl = pl.ds((i * MU + k) * L, L)
            idv[sl] = (idv[sl] + one_i) * mkv[sl] - one_i
        return carry

    lax.fori_loop(0, FLAT // (L * MU), _prep, 0)

    iot = lax.iota(jnp.int32, L)
    thresh = jnp.full((L,), SEQ % L, jnp.int32)
    lane = jnp.where(iot < thresh, one_i, zi)
    one_f = jnp.full((L,), 1.0, jnp.float32)
    zero_f = jnp.zeros((L,), jnp.float32)

    def _zero_slot(j, n):
        def _zb(i, carry):
            s0 = i * 8
            for k in range(8):
                ring[j, s0 + k, pl.ds(0, L)] = zero_f
                ring[j, s0 + k, pl.ds(L, L)] = zero_f
            return carry

        lax.fori_loop(0, n // 8, _zb, 0)

    def _start_chunk(rb, parity, j):
        if parity == 0:
            idx = plsc.Indices(idv.at[pl.ds(rb * SEQ, C1)],
                               ignored_value=SENT)
            return pltpu.async_copy(table_hbm.at[idx], ring.at[j], sems[j])
        idx = plsc.Indices(idv.at[pl.ds(rb * SEQ + C1, C2)],
                           ignored_value=SENT)
        return pltpu.async_copy(table_hbm.at[idx],
                                ring.at[j, pl.ds(0, C2), :], sems[j])

    def _accum(j, n, a0, a1):
        def _body(i, carry):
            b0, b1, b2, b3 = carry
            s0 = i * 8
            for k in range(8):
                lo = ring[j, s0 + k, pl.ds(0, L)]
                hi = ring[j, s0 + k, pl.ds(L, L)]
                if k % 2 == 0:
                    b0 = b0 + lo
                    b1 = b1 + hi
                else:
                    b2 = b2 + lo
                    b3 = b3 + hi
            return (b0, b1, b2, b3)

        b0, b1, b2, b3 = lax.fori_loop(0, n // 8, _body,
                                       (zero_f, zero_f, zero_f, zero_f))
        return a0 + b0 + b2, a1 + b1 + b3

    def _finalize(rb, a0, a1):
        off = rb * SEQ
        cvec = mkv[pl.ds(off + (SEQ // L) * L, L)] * lane
        for k in range(SEQ // L):
            cvec = cvec + mkv[pl.ds(off + k * L, L)]
        for sh in (8, 4, 2, 1):
            perm = iot ^ jnp.full((L,), sh, jnp.int32)
            cvec = cvec + cvec.at[perm].get(mode="promise_in_bounds")
        inv = one_f / jnp.maximum(cvec.astype(jnp.float32), one_f)
        pooled_v[rb, pl.ds(0, L)] = a0 * inv
        pooled_v[rb, pl.ds(L, L)] = a1 * inv

    for j in range(NBUF):
        _zero_slot(j, C1 if j % 2 == 0 else C2)
    handles = [_start_chunk(j // 2, j % 2, j) for j in range(NBUF)]

    def _outer(g, carry):
        c0 = g * NBUF
        for j in range(0, NBUF, 2):
            rb = c0 // 2 + j // 2
            handles[j].wait()
            a0, a1 = _accum(j, C1, zero_f, zero_f)

            @pl.when(c0 + NBUF + j < NCHUNK)
            def _():
                _zero_slot(j, C1)
                _start_chunk((c0 + NBUF + j) // 2, 0, j)

            handles[j + 1].wait()
            a0, a1 = _accum(j + 1, C2, a0, a1)

            @pl.when(c0 + NBUF + j + 1 < NCHUNK)
            def _():
                _zero_slot(j + 1, C2)
                _start_chunk((c0 + NBUF + j + 1) // 2, 1, j + 1)

            _finalize(rb, a0, a1)
        return carry

    lax.fori_loop(0, NCHUNK // NBUF, _outer, 0)

    pltpu.sync_copy(pooled_v, out_hbm.at[pl.ds(wid * BPW, BPW), :])


def _head_body(p_ref, w_ref, b_ref, o_ref):
    o_ref[...] = (
        jnp.dot(p_ref[...], w_ref[...], preferred_element_type=jnp.float32)
        + b_ref[...]
    )


_head = pl.pallas_call(
    _head_body,
    out_shape=jax.ShapeDtypeStruct((BATCH, NUM_LABELS), jnp.float32),
)


def kernel(input_ids, attention_mask, table, W, b):
    ids = input_ids.reshape(-1).astype(jnp.int32)
    mask = attention_mask.reshape(-1).astype(jnp.int32)
    pooled = _sc_pool(ids, mask, table)
    logits = _head(pooled, W, b.reshape(1, NUM_LABELS))
    return (logits, pooled)
